# baseline (speedup 1.0000x reference)
import jax
import jax.numpy as jnp
from jax import lax
from jax.experimental import pallas as pl
from jax.experimental.pallas import tpu as pltpu
from jax.experimental.pallas import tpu_sc as plsc

VOCAB_SIZE = 1000000
HDIM = 64
B, L = 4096, 200

_PAD = 128
_NW = 32
_TOTAL = B * L
_PER_W = _TOTAL // _NW
_CHUNK = 128
_NCHUNK = _PER_W // _CHUNK
_NBUF = 8
_K = 4


def _embed_body(idx_hbm, table_hbm, out_hbm, idx_v, rows_v, g_sems, w_sems):
    wid = lax.axis_index("s") * 2 + lax.axis_index("c")
    base = wid * _PER_W
    pltpu.sync_copy(idx_hbm.at[pl.ds(wid * _NCHUNK, _NCHUNK)], idx_v)

    for b in range(_K):
        pltpu.async_copy(table_hbm.at[idx_v.at[b]], rows_v.at[b], g_sems.at[b])

    def body(j, carry):
        buf = j % _NBUF
        fbuf = (j + _K) % _NBUF

        @pl.when(j + _K < _NCHUNK)
        def _():
            @pl.when(j + _K >= _NBUF)
            def _():
                pltpu.make_async_copy(
                    rows_v.at[fbuf],
                    out_hbm.at[pl.ds(base, _CHUNK), pl.ds(0, HDIM)],
                    w_sems.at[fbuf],
                ).wait()

            pltpu.async_copy(
                table_hbm.at[idx_v.at[j + _K]], rows_v.at[fbuf], g_sems.at[fbuf]
            )

        pltpu.make_async_copy(
            table_hbm.at[idx_v.at[j]], rows_v.at[buf], g_sems.at[buf]
        ).wait()
        pltpu.async_copy(
            rows_v.at[buf],
            out_hbm.at[pl.ds(base + j * _CHUNK, _CHUNK), pl.ds(0, HDIM)],
            w_sems.at[buf],
        )
        return carry

    lax.fori_loop(0, _NCHUNK, body, 0, unroll=False)

    for b in range(_NBUF):
        pltpu.make_async_copy(
            rows_v.at[b], out_hbm.at[pl.ds(base, _CHUNK), pl.ds(0, HDIM)],
            w_sems.at[b],
        ).wait()


@jax.jit
def kernel(x, weight):
    idx = x.reshape(_NW * _NCHUNK, _CHUNK).astype(jnp.int32)
    mesh = plsc.VectorSubcoreMesh(core_axis_name="c", subcore_axis_name="s")
    out = pl.kernel(
        _embed_body,
        mesh=mesh,
        compiler_params=pltpu.CompilerParams(use_tc_tiling_on_sc=False),
        out_type=jax.ShapeDtypeStruct((_TOTAL, _PAD), jnp.float32),
        scratch_types=[
            pltpu.VMEM((_NCHUNK, _CHUNK), jnp.int32),
            pltpu.VMEM((_NBUF, _CHUNK, HDIM), jnp.float32),
            pltpu.SemaphoreType.DMA((_NBUF,)),
            pltpu.SemaphoreType.DMA((_NBUF,)),
        ],
    )(idx, weight)
    return out[:, :HDIM].reshape(B, L, HDIM)
